# baseline (speedup 1.0000x reference)
import jax
import jax.numpy as jnp
from jax.experimental import pallas as pl
from jax.experimental.pallas import tpu as pltpu


def _gcn_block_kernel(feat_ref, w_ref, adj_l_ref, adj_r_ref, out_ref, xw_ref):
    i = pl.program_id(0)

    @pl.when(i == 0)
    def _():
        xw = jnp.dot(feat_ref[...], w_ref[...],
                     preferred_element_type=jnp.float32)
        xw_ref[...] = xw.astype(jnp.bfloat16)

    h = xw_ref.shape[0] // 2
    acc = jnp.dot(adj_l_ref[...].astype(jnp.bfloat16), xw_ref[:h, :],
                  preferred_element_type=jnp.float32)
    acc += jnp.dot(adj_r_ref[...].astype(jnp.bfloat16), xw_ref[h:, :],
                   preferred_element_type=jnp.float32)
    out_ref[...] = jnp.maximum(acc, 0.0)


def kernel(feat, adj, weight):
    n, d_in = feat.shape
    d_out = weight.shape[1]
    bm = 256
    return pl.pallas_call(
        _gcn_block_kernel,
        grid=(n // bm,),
        in_specs=[
            pl.BlockSpec((n, d_in), lambda i: (0, 0)),
            pl.BlockSpec((d_in, d_out), lambda i: (0, 0)),
            pl.BlockSpec((bm, n // 2), lambda i: (i, 0)),
            pl.BlockSpec((bm, n // 2), lambda i: (i, 1)),
        ],
        out_specs=pl.BlockSpec((bm, d_out), lambda i: (i, 0)),
        out_shape=jax.ShapeDtypeStruct((n, d_out), jnp.float32),
        scratch_shapes=[pltpu.VMEM((n, d_out), jnp.bfloat16)],
    )(feat, weight, adj, adj)
